# baseline (speedup 1.0000x reference)
import jax
import jax.numpy as jnp
from jax import lax
from jax.experimental import pallas as pl
from jax.experimental.pallas import tpu as pltpu
from jax.experimental.pallas import tpu_sc as plsc

_NC = 2
_NS = 16
_NW = _NC * _NS
_CHUNK = 256


def kernel(x, table):
    b, s = x.shape
    n = b * s
    v, d = table.shape
    dw = 2 * d
    idx = x.reshape(n)
    per_w = n // _NW

    nk = 8
    vc = v // nk
    tab_e = jnp.concatenate(
        [
            jnp.pad(
                jax.lax.optimization_barrier(table[k * vc:(k + 1) * vc]),
                ((0, 0), (0, d)),
            )
            for k in range(nk)
        ],
        axis=0,
    )

    mesh = plsc.VectorSubcoreMesh(core_axis_name="c", subcore_axis_name="s")

    @pl.kernel(
        out_type=jax.ShapeDtypeStruct((n, dw), table.dtype),
        mesh=mesh,
        scratch_types=[
            pltpu.VMEM((per_w,), jnp.int32),
            pltpu.VMEM((_CHUNK, dw), table.dtype),
            pltpu.VMEM((_CHUNK, dw), table.dtype),
            pltpu.SemaphoreType.DMA,
            pltpu.SemaphoreType.DMA,
            pltpu.SemaphoreType.DMA,
            pltpu.SemaphoreType.DMA,
        ],
    )
    def gather_kernel(
        tab_hbm, idx_hbm, out_hbm, idx_v, buf0, buf1, sg0, sg1, sw0, sw1
    ):
        wid = lax.axis_index("s") * _NC + lax.axis_index("c")
        base = wid * per_w
        pltpu.sync_copy(idx_hbm.at[pl.ds(base, per_w)], idx_v)

        @pl.loop(0, per_w, step=2 * _CHUNK)
        def _(off):
            g0 = pltpu.async_copy(
                tab_hbm.at[idx_v.at[pl.ds(off, _CHUNK)]], buf0, sg0
            )
            g1 = pltpu.async_copy(
                tab_hbm.at[idx_v.at[pl.ds(off + _CHUNK, _CHUNK)]], buf1, sg1
            )
            g0.wait()
            w0 = pltpu.async_copy(
                buf0, out_hbm.at[pl.ds(base + off, _CHUNK)], sw0
            )
            g1.wait()
            w1 = pltpu.async_copy(
                buf1, out_hbm.at[pl.ds(base + off + _CHUNK, _CHUNK)], sw1
            )
            w0.wait()
            w1.wait()

    wide = gather_kernel(tab_e, idx)
    return wide[:, :d].reshape(b, s, d)


# speedup vs baseline: 1.3001x; 1.3001x over previous
import jax
import jax.numpy as jnp
from jax import lax
from jax.experimental import pallas as pl
from jax.experimental.pallas import tpu as pltpu
from jax.experimental.pallas import tpu_sc as plsc

_NC = 2
_NS = 16
_NW = _NC * _NS
_CHUNK = 256


def kernel(x, table):
    b, s = x.shape
    n = b * s
    v, d = table.shape
    dw = 2 * d
    idx = x.reshape(n)
    per_w = n // _NW

    tab_e = jnp.pad(table, ((0, 0), (0, d)))

    mesh = plsc.VectorSubcoreMesh(core_axis_name="c", subcore_axis_name="s")

    @pl.kernel(
        out_type=jax.ShapeDtypeStruct((n, dw), table.dtype),
        mesh=mesh,
        scratch_types=[
            pltpu.VMEM((per_w,), jnp.int32),
            pltpu.VMEM((_CHUNK, dw), table.dtype),
            pltpu.VMEM((_CHUNK, dw), table.dtype),
            pltpu.SemaphoreType.DMA,
            pltpu.SemaphoreType.DMA,
            pltpu.SemaphoreType.DMA,
            pltpu.SemaphoreType.DMA,
        ],
    )
    def gather_kernel(
        tab_hbm, idx_hbm, out_hbm, idx_v, buf0, buf1, sg0, sg1, sw0, sw1
    ):
        wid = lax.axis_index("s") * _NC + lax.axis_index("c")
        base = wid * per_w
        pltpu.sync_copy(idx_hbm.at[pl.ds(base, per_w)], idx_v)

        @pl.loop(0, per_w, step=2 * _CHUNK)
        def _(off):
            g0 = pltpu.async_copy(
                tab_hbm.at[idx_v.at[pl.ds(off, _CHUNK)]], buf0, sg0
            )
            g1 = pltpu.async_copy(
                tab_hbm.at[idx_v.at[pl.ds(off + _CHUNK, _CHUNK)]], buf1, sg1
            )
            g0.wait()
            w0 = pltpu.async_copy(
                buf0, out_hbm.at[pl.ds(base + off, _CHUNK)], sw0
            )
            g1.wait()
            w1 = pltpu.async_copy(
                buf1, out_hbm.at[pl.ds(base + off + _CHUNK, _CHUNK)], sw1
            )
            w0.wait()
            w1.wait()

    wide = gather_kernel(tab_e, idx)
    return wide[:, :d].reshape(b, s, d)


# speedup vs baseline: 1.3061x; 1.0046x over previous
import jax
import jax.numpy as jnp
from jax import lax
from jax.experimental import pallas as pl
from jax.experimental.pallas import tpu as pltpu
from jax.experimental.pallas import tpu_sc as plsc

_NC = 2
_NS = 16
_NW = _NC * _NS
_CHUNK = 400


def kernel(x, table):
    b, s = x.shape
    n = b * s
    v, d = table.shape
    dw = 2 * d
    idx = x.reshape(n)
    per_w = n // _NW

    tab_e = jnp.pad(table, ((0, 0), (0, d)))

    mesh = plsc.VectorSubcoreMesh(core_axis_name="c", subcore_axis_name="s")

    @pl.kernel(
        out_type=jax.ShapeDtypeStruct((n, dw), table.dtype),
        mesh=mesh,
        scratch_types=[
            pltpu.VMEM((per_w,), jnp.int32),
            pltpu.VMEM((_CHUNK, dw), table.dtype),
            pltpu.VMEM((_CHUNK, dw), table.dtype),
            pltpu.SemaphoreType.DMA,
            pltpu.SemaphoreType.DMA,
            pltpu.SemaphoreType.DMA,
            pltpu.SemaphoreType.DMA,
        ],
    )
    def gather_kernel(
        tab_hbm, idx_hbm, out_hbm, idx_v, buf0, buf1, sg0, sg1, sw0, sw1
    ):
        wid = lax.axis_index("s") * _NC + lax.axis_index("c")
        base = wid * per_w
        pltpu.sync_copy(idx_hbm.at[pl.ds(base, per_w)], idx_v)

        @pl.loop(0, per_w, step=2 * _CHUNK)
        def _(off):
            g0 = pltpu.async_copy(
                tab_hbm.at[idx_v.at[pl.ds(off, _CHUNK)]], buf0, sg0
            )
            g1 = pltpu.async_copy(
                tab_hbm.at[idx_v.at[pl.ds(off + _CHUNK, _CHUNK)]], buf1, sg1
            )
            g0.wait()
            w0 = pltpu.async_copy(
                buf0, out_hbm.at[pl.ds(base + off, _CHUNK)], sw0
            )
            g1.wait()
            w1 = pltpu.async_copy(
                buf1, out_hbm.at[pl.ds(base + off + _CHUNK, _CHUNK)], sw1
            )
            w0.wait()
            w1.wait()

    wide = gather_kernel(tab_e, idx)
    return wide[:, :d].reshape(b, s, d)
